# baseline (speedup 1.0000x reference)
import functools

import jax
import jax.numpy as jnp
from jax import lax
from jax.experimental import pallas as pl
from jax.experimental.pallas import tpu as pltpu
from jax.experimental.pallas import tpu_sc as plsc

B = 4096
KEY_D = 768
EMB_D = 768
POOL = 1000
PPAD = 1024
E_P_LEN = 8
HALF_LEN = E_P_LEN // 2

N_SLICES = 2
BS = B // N_SLICES
BQ = 1024
XB_BLK = B // (BS // BQ)


def _argmax_rows(s):
    col = lax.broadcasted_iota(jnp.int32, s.shape, 1)
    s = jnp.where(col < POOL, s, -jnp.inf)
    m = jnp.max(s, axis=1, keepdims=True)
    return jnp.min(jnp.where(s == m, col, PPAD), axis=1)


def _topk_body(q_ref, ek_ref, idx_ref):
    s = lax.dot_general(q_ref[...], ek_ref[...], (((1,), (1,)), ((), ())),
                        preferred_element_type=jnp.float32)
    idx_ref[...] = _argmax_rows(s).astype(jnp.int32).reshape(1, 1, BQ)


def _topk_xb_body(q_ref, ek_ref, xb_ref, idx_ref, xb_out_ref):
    s = lax.dot_general(q_ref[...], ek_ref[...], (((1,), (1,)), ((), ())),
                        preferred_element_type=jnp.float32)
    idx_ref[...] = _argmax_rows(s).astype(jnp.int32).reshape(1, 1, BQ)
    xb_out_ref[...] = xb_ref[...]


def _topk_call(qn, e_k_pad, sl):
    nb = BS // BQ
    off = sl * nb
    out = pl.pallas_call(
        _topk_body,
        grid=(nb,),
        in_specs=[
            pl.BlockSpec((BQ, KEY_D), lambda i: (i + off, 0)),
            pl.BlockSpec((PPAD, KEY_D), lambda i: (0, 0)),
        ],
        out_specs=pl.BlockSpec((1, 1, BQ), lambda i: (i, 0, 0)),
        out_shape=jax.ShapeDtypeStruct((nb, 1, BQ), jnp.int32),
    )(qn, e_k_pad)
    return out.reshape(BS)


def _topk_xb_call(qn, e_k_pad, x_block, sl):
    nb = BS // BQ
    off = sl * nb
    idx, xb = pl.pallas_call(
        _topk_xb_body,
        grid=(nb,),
        in_specs=[
            pl.BlockSpec((BQ, KEY_D), lambda i: (i + off, 0)),
            pl.BlockSpec((PPAD, KEY_D), lambda i: (0, 0)),
            pl.BlockSpec((XB_BLK, EMB_D), lambda i: (i, 0)),
        ],
        out_specs=[
            pl.BlockSpec((1, 1, BQ), lambda i: (i, 0, 0)),
            pl.BlockSpec((XB_BLK, EMB_D), lambda i: (i, 0)),
        ],
        out_shape=[
            jax.ShapeDtypeStruct((nb, 1, BQ), jnp.int32),
            jax.ShapeDtypeStruct((B, EMB_D), jnp.float32),
        ],
    )(qn, e_k_pad, x_block)
    return idx.reshape(BS), xb


def _make_gather(sl, alloc):
    info = plsc.get_sparse_core_info()
    nc, ns = info.num_cores, info.num_subcores
    nw = nc * ns
    b_per_w = BS // nw
    chunk = 8
    n_chunks = b_per_w // chunk
    out_off = sl * BS
    mesh = plsc.VectorSubcoreMesh(core_axis_name="c", subcore_axis_name="s")
    out_sds = jax.ShapeDtypeStruct((B, HALF_LEN, EMB_D), jnp.float32)

    @functools.partial(
        pl.kernel,
        mesh=mesh,
        compiler_params=pltpu.CompilerParams(use_tc_tiling_on_sc=True),
        out_type=(out_sds, out_sds) if alloc else (),
        scratch_types=[
            pltpu.VMEM((b_per_w,), jnp.int32),
            pltpu.VMEM((2, chunk, E_P_LEN, EMB_D), jnp.float32),
            pltpu.SemaphoreType.DMA,
            pltpu.SemaphoreType.DMA,
            pltpu.SemaphoreType.DMA,
            pltpu.SemaphoreType.DMA,
        ],
    )
    def gather(table_hbm, idx_hbm, ek_hbm, ev_hbm, idx_v, rows_v,
               in_s0, in_s1, out_s0, out_s1):
        wid = lax.axis_index("s") * nc + lax.axis_index("c")
        base = wid * b_per_w
        in_sems = (in_s0, in_s1)
        out_sems = (out_s0, out_s1)
        pltpu.sync_copy(idx_hbm.at[pl.ds(base, b_per_w)], idx_v)

        def start_in(c):
            b = c & 1
            return pltpu.async_copy(
                table_hbm.at[idx_v.at[pl.ds(c * chunk, chunk)]],
                rows_v.at[b], in_sems[b])

        def start_out(c):
            b = c & 1
            dst = pl.ds(out_off + base + c * chunk, chunk)
            return (
                pltpu.async_copy(rows_v.at[b, :, pl.ds(0, HALF_LEN)],
                                 ek_hbm.at[dst], out_sems[b]),
                pltpu.async_copy(rows_v.at[b, :, pl.ds(HALF_LEN, HALF_LEN)],
                                 ev_hbm.at[dst], out_sems[b]),
            )

        pend_in = {0: start_in(0)}
        pend_out = {}
        for c in range(n_chunks):
            if c + 1 < n_chunks:
                if c - 1 in pend_out:
                    for h in pend_out.pop(c - 1):
                        h.wait()
                pend_in[c + 1] = start_in(c + 1)
            pend_in.pop(c).wait()
            pend_out[c] = start_out(c)
        for c in sorted(pend_out):
            for h in pend_out.pop(c):
                h.wait()

    return gather


_gathers = None


def _get_gathers():
    global _gathers
    if _gathers is None:
        _gathers = tuple(
            _make_gather(sl, alloc=(sl == 0)) for sl in range(N_SLICES))
    return _gathers


def _normalize(x, axis):
    n = jnp.linalg.norm(x, axis=axis, keepdims=True)
    return x / jnp.clip(n, 1e-12)


def kernel(x_querry, l, x_block, e_k, e_p):
    n_K = _normalize(e_k, 1)
    qn = jax.lax.stop_gradient(_normalize(x_querry, 1))
    e_k_pad = jnp.pad(n_K, ((0, PPAD - POOL), (0, 0)))
    gathers = _get_gathers()

    idx0 = _topk_call(qn, e_k_pad, 0)
    ek_buf, ev_buf = gathers[0](e_p, idx0)
    ek_ref, ev_ref = jax.new_ref(ek_buf), jax.new_ref(ev_buf)
    idx1, xb = _topk_xb_call(qn, e_k_pad, x_block, 1)
    gathers[1](e_p, idx1, ek_ref, ev_ref)
    return (ek_ref[...], ev_ref[...], xb)


# speedup vs baseline: 1.0338x; 1.0338x over previous
import functools

import jax
import jax.numpy as jnp
from jax import lax
from jax.experimental import pallas as pl
from jax.experimental.pallas import tpu as pltpu
from jax.experimental.pallas import tpu_sc as plsc

B = 4096
KEY_D = 768
EMB_D = 768
POOL = 1000
PPAD = 1024
E_P_LEN = 8
HALF_LEN = E_P_LEN // 2

N_SLICES = 2
BS = B // N_SLICES
BQ = 512
def _argmax_rows(s):
    col = lax.broadcasted_iota(jnp.int32, s.shape, 1)
    s = jnp.where(col < POOL, s, -jnp.inf)
    m = jnp.max(s, axis=1, keepdims=True)
    return jnp.min(jnp.where(s == m, col, PPAD), axis=1)


def _topk_body(q_ref, ek_ref, idx_ref):
    s = lax.dot_general(q_ref[...], ek_ref[...], (((1,), (1,)), ((), ())),
                        preferred_element_type=jnp.float32)
    idx_ref[...] = _argmax_rows(s).astype(jnp.int32).reshape(1, 1, BQ)


def _xb_body(xb_ref, idx_ref, out_ref):
    del idx_ref
    out_ref[...] = xb_ref[...]


def _topk_call(qn, e_k_pad, sl):
    nb = BS // BQ
    off = sl * nb
    out = pl.pallas_call(
        _topk_body,
        grid=(nb,),
        in_specs=[
            pl.BlockSpec((BQ, KEY_D), lambda i: (i + off, 0)),
            pl.BlockSpec((PPAD, KEY_D), lambda i: (0, 0)),
        ],
        out_specs=pl.BlockSpec((1, 1, BQ), lambda i: (i, 0, 0)),
        out_shape=jax.ShapeDtypeStruct((nb, 1, BQ), jnp.int32),
    )(qn, e_k_pad)
    return out


def _xb_call(x_block, idx):
    nb = 4
    blk = B // nb
    return pl.pallas_call(
        _xb_body,
        grid=(nb,),
        in_specs=[
            pl.BlockSpec((blk, EMB_D), lambda i: (i, 0)),
            pl.BlockSpec((1, 1, BQ), lambda i: (0, 0, 0)),
        ],
        out_specs=pl.BlockSpec((blk, EMB_D), lambda i: (i, 0)),
        out_shape=jax.ShapeDtypeStruct((B, EMB_D), jnp.float32),
    )(x_block, idx)


def _make_gather(sl, alloc):
    info = plsc.get_sparse_core_info()
    nc, ns = info.num_cores, info.num_subcores
    nw = nc * ns
    b_per_w = BS // nw
    chunk = 8
    n_chunks = b_per_w // chunk
    out_off = sl * BS
    mesh = plsc.VectorSubcoreMesh(core_axis_name="c", subcore_axis_name="s")
    out_sds = jax.ShapeDtypeStruct((B, HALF_LEN, EMB_D), jnp.float32)

    @functools.partial(
        pl.kernel,
        mesh=mesh,
        compiler_params=pltpu.CompilerParams(use_tc_tiling_on_sc=True),
        out_type=(out_sds, out_sds) if alloc else (),
        scratch_types=[
            pltpu.VMEM((b_per_w,), jnp.int32),
            pltpu.VMEM((2, chunk, E_P_LEN, EMB_D), jnp.float32),
            pltpu.SemaphoreType.DMA,
            pltpu.SemaphoreType.DMA,
            pltpu.SemaphoreType.DMA,
            pltpu.SemaphoreType.DMA,
        ],
    )
    def gather(table_hbm, idx_hbm, ek_hbm, ev_hbm, idx_v, rows_v,
               in_s0, in_s1, out_s0, out_s1):
        wid = lax.axis_index("s") * nc + lax.axis_index("c")
        base = wid * b_per_w
        in_sems = (in_s0, in_s1)
        out_sems = (out_s0, out_s1)
        pltpu.sync_copy(idx_hbm.at[pl.ds(base, b_per_w)], idx_v)

        def start_in(c):
            b = c & 1
            return pltpu.async_copy(
                table_hbm.at[idx_v.at[pl.ds(c * chunk, chunk)]],
                rows_v.at[b], in_sems[b])

        def start_out(c):
            b = c & 1
            dst = pl.ds(out_off + base + c * chunk, chunk)
            return (
                pltpu.async_copy(rows_v.at[b, :, pl.ds(0, HALF_LEN)],
                                 ek_hbm.at[dst], out_sems[b]),
                pltpu.async_copy(rows_v.at[b, :, pl.ds(HALF_LEN, HALF_LEN)],
                                 ev_hbm.at[dst], out_sems[b]),
            )

        pend_in = {0: start_in(0)}
        pend_out = {}
        for c in range(n_chunks):
            if c + 1 < n_chunks:
                if c - 1 in pend_out:
                    for h in pend_out.pop(c - 1):
                        h.wait()
                pend_in[c + 1] = start_in(c + 1)
            pend_in.pop(c).wait()
            pend_out[c] = start_out(c)
        for c in sorted(pend_out):
            for h in pend_out.pop(c):
                h.wait()

    return gather


_gathers = None


def _get_gathers():
    global _gathers
    if _gathers is None:
        _gathers = tuple(
            _make_gather(sl, alloc=(sl == 0)) for sl in range(N_SLICES))
    return _gathers


def _normalize(x, axis):
    n = jnp.linalg.norm(x, axis=axis, keepdims=True)
    return x / jnp.clip(n, 1e-12)


def kernel(x_querry, l, x_block, e_k, e_p):
    n_K = _normalize(e_k, 1)
    qn = jax.lax.stop_gradient(_normalize(x_querry, 1))
    e_k_pad = jnp.pad(n_K, ((0, PPAD - POOL), (0, 0)))
    gathers = _get_gathers()

    idx0 = _topk_call(qn, e_k_pad, 0)
    ek_buf, ev_buf = gathers[0](e_p, idx0.reshape(BS))
    ek_ref, ev_ref = jax.new_ref(ek_buf), jax.new_ref(ev_buf)
    idx1 = _topk_call(qn, e_k_pad, 1)
    xb = _xb_call(x_block, idx1)
    gathers[1](e_p, idx1.reshape(BS), ek_ref, ev_ref)
    return (ek_ref[...], ev_ref[...], xb)
